# baseline (speedup 1.0000x reference)
import functools

import jax
import jax.numpy as jnp
from jax import lax
from jax.experimental import pallas as pl
from jax.experimental.pallas import tpu as pltpu
from jax.experimental.pallas import tpu_sc as plsc

EMB = 128
NC = 2
NS = 16
NW = NC * NS
NBUF = 10
AHEAD = 5
HALF = 2


@functools.lru_cache(maxsize=None)
def _make_gather(N, S):
    n_per_w = N // NW
    c_sent = n_per_w // HALF
    nchunk = S * HALF
    ngroup = nchunk // NBUF
    assert nchunk % NBUF == 0 and ngroup >= 3
    mesh = plsc.VectorSubcoreMesh(core_axis_name="c", subcore_axis_name="s")

    @functools.partial(
        pl.kernel,
        mesh=mesh,
        out_type=jax.ShapeDtypeStruct((S, N, EMB), jnp.float32),
        scratch_types=[
            pltpu.VMEM((nchunk, c_sent), jnp.int32),
            pltpu.VMEM((NBUF, c_sent, EMB), jnp.float32),
        ]
        + [pltpu.SemaphoreType.DMA] * (2 * NBUF),
    )
    def gather_kernel(idx_hbm, table_hbm, out_hbm, idx_v, rows_v, *sems):
        gsems, ssems = sems[:NBUF], sems[NBUF:]
        wid = lax.axis_index("s") * NC + lax.axis_index("c")
        pltpu.sync_copy(idx_hbm.at[wid], idx_v)
        base = wid * n_per_w

        def out_slice(i):
            return out_hbm.at[i // HALF, pl.ds(base + (i % HALF) * c_sent, c_sent)]

        def issue_gather(i, j):
            pltpu.async_copy(table_hbm.at[idx_v.at[i]], rows_v.at[j], gsems[j])

        def wait_gather(j):
            pltpu.make_async_copy(
                table_hbm.at[idx_v.at[0]], rows_v.at[j], gsems[j]
            ).wait()

        def issue_store(i, j):
            pltpu.async_copy(rows_v.at[j], out_slice(i), ssems[j])

        def wait_store(i, j):
            pltpu.make_async_copy(rows_v.at[j], out_slice(i), ssems[j]).wait()

        def step(i, b, do_wait, do_issue):
            s = (b + AHEAD) % NBUF
            if do_wait:
                wait_store(i + AHEAD - NBUF, s)
            if do_issue:
                issue_gather(i + AHEAD, s)
            wait_gather(b)
            issue_store(i, b)

        for j in range(AHEAD):
            issue_gather(j, j)

        for b in range(NBUF):
            step(b, b, do_wait=b + AHEAD >= NBUF, do_issue=True)

        def group_body(g, carry):
            for b in range(NBUF):
                step(g * NBUF + b, b, do_wait=True, do_issue=True)
            return carry

        lax.fori_loop(1, ngroup - 1, group_body, 0)

        for b in range(NBUF):
            i = (ngroup - 1) * NBUF + b
            step(i, b, do_wait=True, do_issue=i + AHEAD < nchunk)
        for i in range(nchunk - NBUF + AHEAD, nchunk):
            wait_store(i, i % NBUF)

    return gather_kernel


def kernel(input_tensor, table):
    n, s = input_tensor.shape
    idx = (input_tensor.reshape(NW, n // NW, s).transpose(0, 2, 1)
           .reshape(NW, s * HALF, (n // NW) // HALF))
    out = _make_gather(n, s)(idx, table)
    return out.transpose(1, 0, 2)


# speedup vs baseline: 1.0124x; 1.0124x over previous
import functools

import jax
import jax.numpy as jnp
from jax import lax
from jax.experimental import pallas as pl
from jax.experimental.pallas import tpu as pltpu
from jax.experimental.pallas import tpu_sc as plsc

EMB = 128
NC = 2
NS = 16
NW = NC * NS
NBUF = 5
AHEAD = 2


@functools.lru_cache(maxsize=None)
def _make_gather(N, S):
    n_per_w = N // NW
    nchunk = S
    ngroup = nchunk // NBUF
    assert nchunk % NBUF == 0 and ngroup >= 3
    mesh = plsc.VectorSubcoreMesh(core_axis_name="c", subcore_axis_name="s")

    @functools.partial(
        pl.kernel,
        mesh=mesh,
        out_type=jax.ShapeDtypeStruct((S, N, EMB), jnp.float32),
        scratch_types=[
            pltpu.VMEM((S, n_per_w), jnp.int32),
            pltpu.VMEM((NBUF, n_per_w, EMB), jnp.float32),
        ]
        + [pltpu.SemaphoreType.DMA] * (2 * NBUF),
    )
    def gather_kernel(idx_hbm, table_hbm, out_hbm, idx_v, rows_v, *sems):
        gsems, ssems = sems[:NBUF], sems[NBUF:]
        wid = lax.axis_index("s") * NC + lax.axis_index("c")
        base = wid * n_per_w
        pltpu.sync_copy(idx_hbm.at[pl.ds(0, S), pl.ds(base, n_per_w)], idx_v)

        def issue_gather(i, j):
            pltpu.async_copy(table_hbm.at[idx_v.at[i]], rows_v.at[j], gsems[j])

        def wait_gather(j):
            pltpu.make_async_copy(
                table_hbm.at[idx_v.at[0]], rows_v.at[j], gsems[j]
            ).wait()

        def issue_store(i, j):
            pltpu.async_copy(
                rows_v.at[j], out_hbm.at[i, pl.ds(base, n_per_w)], ssems[j]
            )

        def wait_store(i, j):
            pltpu.make_async_copy(
                rows_v.at[j], out_hbm.at[i, pl.ds(base, n_per_w)], ssems[j]
            ).wait()

        def step(i, b, do_wait, do_issue):
            s = (b + AHEAD) % NBUF
            if do_wait:
                wait_store(i + AHEAD - NBUF, s)
            if do_issue:
                issue_gather(i + AHEAD, s)
            wait_gather(b)
            issue_store(i, b)

        for j in range(AHEAD):
            issue_gather(j, j)

        for b in range(NBUF):
            step(b, b, do_wait=b + AHEAD >= NBUF, do_issue=True)

        def group_body(g, carry):
            for b in range(NBUF):
                step(g * NBUF + b, b, do_wait=True, do_issue=True)
            return carry

        lax.fori_loop(1, ngroup - 1, group_body, 0)

        for b in range(NBUF):
            i = (ngroup - 1) * NBUF + b
            step(i, b, do_wait=True, do_issue=i + AHEAD < nchunk)
        for i in range(nchunk - NBUF + AHEAD, nchunk):
            wait_store(i, i % NBUF)

    return gather_kernel


def kernel(input_tensor, table):
    n, s = input_tensor.shape
    out = _make_gather(n, s)(input_tensor.T, table)
    return out.transpose(1, 0, 2)
